# baseline (speedup 1.0000x reference)
import jax
import jax.numpy as jnp
import numpy as np
from jax.experimental import pallas as pl

H = W = 112
B = 64
G = 16

RATIOS = ((64, 64), (51, 79), (79, 51), (76, 53), (53, 76))
OUT_HW = tuple((H - kh + 1, W - kw + 1) for kh, kw in RATIOS)
OFFSETS = (0, 2401, 4509, 6617, 8837)
BIG = 2**30


def _band(n, k):
    t = np.arange(n)[:, None]
    j = np.arange(n)[None, :]
    m = (j <= t) & (t < j + k) & (j <= n - k)
    return jnp.asarray(m, dtype=jnp.bfloat16)


def _split(a):
    hi = a.astype(jnp.bfloat16)
    lo = (a - hi.astype(jnp.float32)).astype(jnp.bfloat16)
    return hi, lo


def _dot2(ah, al, b):
    f32 = jnp.float32
    return jnp.dot(ah, b, preferred_element_type=f32) + jnp.dot(
        al, b, preferred_element_type=f32
    )


def _dot2l(a, bh, bl):
    f32 = jnp.float32
    return jnp.dot(a, bh, preferred_element_type=f32) + jnp.dot(
        a, bl, preferred_element_type=f32
    )


def _kernel_body(x_ref, *refs):
    b_refs = refs[:5]
    a_refs = refs[5:10]
    outs = refs[10:15]
    idx_ref, val_ref = refs[15], refs[16]
    xg = x_ref[...].reshape(G * H, W)
    xh, xl = _split(xg)
    for r, (kh, kw) in enumerate(RATIOS):
        oh, ow = OUT_HW[r]
        xw = _dot2(xh, xl, b_refs[r][...]) * (1.0 / float(kh * kw))
        wh, wl = _split(xw)
        amat = a_refs[r][...]
        for b in range(G):
            hs = _dot2l(amat, wh[b * H : (b + 1) * H, :], wl[b * H : (b + 1) * H, :])
            outs[r][b, :, :] = hs[:, :ow]
    best_val = None
    best_idx = None
    for r in range(4):
        oh, ow = OUT_HW[r]
        sc3 = outs[r][...]
        m = jnp.max(sc3, axis=(1, 2))
        flat = (
            jax.lax.broadcasted_iota(jnp.int32, (G, oh, ow), 1) * ow
            + jax.lax.broadcasted_iota(jnp.int32, (G, oh, ow), 2)
            + OFFSETS[r]
        )
        cand = jnp.min(
            jnp.where(sc3 == m[:, None, None], flat, BIG), axis=(1, 2)
        )
        if best_val is None:
            best_val, best_idx = m, cand
        else:
            take_new = m > best_val
            best_idx = jnp.where(
                take_new,
                cand,
                jnp.where(m == best_val, jnp.minimum(best_idx, cand), best_idx),
            )
            best_val = jnp.maximum(best_val, m)
    idx_ref[...] = best_idx.reshape(G, 1)
    val_ref[...] = best_val.reshape(G, 1)


@jax.jit
def _run(x3, *mats):
    grid = B // G
    out_shapes = [
        jax.ShapeDtypeStruct((B, oh, ow), jnp.float32) for oh, ow in OUT_HW
    ] + [
        jax.ShapeDtypeStruct((B, 1), jnp.int32),
        jax.ShapeDtypeStruct((B, 1), jnp.float32),
    ]
    out_specs = [
        pl.BlockSpec((G, oh, ow), lambda i: (i, 0, 0)) for oh, ow in OUT_HW
    ] + [
        pl.BlockSpec((G, 1), lambda i: (i, 0)),
        pl.BlockSpec((G, 1), lambda i: (i, 0)),
    ]
    in_specs = (
        [pl.BlockSpec((G, H, W), lambda i: (i, 0, 0))]
        + [pl.BlockSpec((H, W), lambda i: (0, 0)) for _ in range(5)]
        + [pl.BlockSpec((oh, W), lambda i: (0, 0)) for oh, _ in OUT_HW]
    )
    return pl.pallas_call(
        _kernel_body,
        grid=(grid,),
        in_specs=in_specs,
        out_specs=out_specs,
        out_shape=out_shapes,
    )(x3, *mats)


def kernel(x):
    x3 = x.reshape(B, H, W)
    bmats = [_band(W, kw) for _, kw in RATIOS]
    amats = [_band(H, kh).T[: H - kh + 1] for kh, kw in RATIOS]
    *grids, idx, val = _run(x3, *bmats, *amats)
    ws = jnp.concatenate([g.reshape(B, -1) for g in grids], axis=1)
    return (idx, val, ws)


# speedup vs baseline: 1.0026x; 1.0026x over previous
import jax
import jax.numpy as jnp
import numpy as np
from jax.experimental import pallas as pl
from jax.experimental.pallas import tpu as pltpu

H = W = 112
B = 64
G = 32

RATIOS = ((64, 64), (51, 79), (79, 51), (76, 53), (53, 76))
OUT_HW = tuple((H - kh + 1, W - kw + 1) for kh, kw in RATIOS)
OFFSETS = (0, 2401, 4509, 6617, 8837)
BIG = 2**30


def _band(n, k):
    t = np.arange(n)[:, None]
    j = np.arange(n)[None, :]
    m = (j <= t) & (t < j + k) & (j <= n - k)
    return jnp.asarray(m, dtype=jnp.bfloat16)


def _split(a):
    hi = a.astype(jnp.bfloat16)
    lo = (a - hi.astype(jnp.float32)).astype(jnp.bfloat16)
    return hi, lo


def _dot2(ah, al, b):
    f32 = jnp.float32
    return jnp.dot(ah, b, preferred_element_type=f32) + jnp.dot(
        al, b, preferred_element_type=f32
    )


def _dot2l(a, bh, bl):
    f32 = jnp.float32
    return jnp.dot(a, bh, preferred_element_type=f32) + jnp.dot(
        a, bl, preferred_element_type=f32
    )


def _kernel_body(x_ref, *refs):
    b_refs = refs[:5]
    a_refs = refs[5:10]
    outs = refs[10:15]
    idx_ref, val_ref = refs[15], refs[16]
    xg = x_ref[...].reshape(G * H, W)
    xh, xl = _split(xg)
    for r, (kh, kw) in enumerate(RATIOS):
        oh, ow = OUT_HW[r]
        xw = _dot2(xh, xl, b_refs[r][...]) * (1.0 / float(kh * kw))
        wh, wl = _split(xw)
        amat = a_refs[r][...]
        for b in range(G):
            hs = _dot2l(amat, wh[b * H : (b + 1) * H, :], wl[b * H : (b + 1) * H, :])
            outs[r][b, :, :] = hs[:, :ow]
    best_val = None
    best_idx = None
    for r in range(4):
        oh, ow = OUT_HW[r]
        sc3 = outs[r][...]
        m = jnp.max(sc3, axis=(1, 2))
        flat = (
            jax.lax.broadcasted_iota(jnp.int32, (G, oh, ow), 1) * ow
            + jax.lax.broadcasted_iota(jnp.int32, (G, oh, ow), 2)
            + OFFSETS[r]
        )
        cand = jnp.min(
            jnp.where(sc3 == m[:, None, None], flat, BIG), axis=(1, 2)
        )
        if best_val is None:
            best_val, best_idx = m, cand
        else:
            take_new = m > best_val
            best_idx = jnp.where(
                take_new,
                cand,
                jnp.where(m == best_val, jnp.minimum(best_idx, cand), best_idx),
            )
            best_val = jnp.maximum(best_val, m)
    idx_ref[...] = best_idx.reshape(G, 1)
    val_ref[...] = best_val.reshape(G, 1)


@jax.jit
def _run(x3, *mats):
    grid = B // G
    out_shapes = [
        jax.ShapeDtypeStruct((B, oh, ow), jnp.float32) for oh, ow in OUT_HW
    ] + [
        jax.ShapeDtypeStruct((B, 1), jnp.int32),
        jax.ShapeDtypeStruct((B, 1), jnp.float32),
    ]
    out_specs = [
        pl.BlockSpec((G, oh, ow), lambda i: (i, 0, 0)) for oh, ow in OUT_HW
    ] + [
        pl.BlockSpec((G, 1), lambda i: (i, 0)),
        pl.BlockSpec((G, 1), lambda i: (i, 0)),
    ]
    in_specs = (
        [pl.BlockSpec((G, H, W), lambda i: (i, 0, 0))]
        + [pl.BlockSpec((H, W), lambda i: (0, 0)) for _ in range(5)]
        + [pl.BlockSpec((oh, W), lambda i: (0, 0)) for oh, _ in OUT_HW]
    )
    return pl.pallas_call(
        _kernel_body,
        grid=(grid,),
        in_specs=in_specs,
        out_specs=out_specs,
        out_shape=out_shapes,
    )(x3, *mats)


SIZES = tuple(oh * ow for oh, ow in OUT_HW)


def _concat_body(*refs):
    ins, ws_ref, sems = refs[:5], refs[5], refs[6:11]
    copies = []
    off = 0
    for r in range(5):
        sz = SIZES[r]
        c = pltpu.make_async_copy(ins[r], ws_ref.at[:, pl.ds(off, sz)], sems[r])
        c.start()
        copies.append(c)
        off += sz
    for c in copies:
        c.wait()


@jax.jit
def _concat_run(*flat_grids):
    return pl.pallas_call(
        _concat_body,
        in_specs=[pl.BlockSpec(memory_space=pl.ANY) for _ in range(5)],
        out_specs=pl.BlockSpec(memory_space=pl.ANY),
        out_shape=jax.ShapeDtypeStruct((B, sum(SIZES)), jnp.float32),
        scratch_shapes=[pltpu.SemaphoreType.DMA] * 5,
    )(*flat_grids)


def kernel(x):
    x3 = x.reshape(B, H, W)
    bmats = [_band(W, kw) for _, kw in RATIOS]
    amats = [_band(H, kh).T[: H - kh + 1] for kh, kw in RATIOS]
    *grids, idx, val = _run(x3, *bmats, *amats)
    ws = jnp.concatenate([g.reshape(B, -1) for g in grids], axis=1)
    return (idx, val, ws)
